# Initial kernel scaffold; baseline (speedup 1.0000x reference)
import functools

import jax
import jax.numpy as jnp
from jax import lax
from jax.experimental import pallas as pl
from jax.experimental.pallas import tpu as pltpu
from jax.experimental.pallas import tpu_sc as plsc

D_MODEL = 64
NUM_WORKERS = 32
ROWS_PER_GATHER = 128
GATHERS_PER_GROUP = 8
GROUP = ROWS_PER_GATHER * GATHERS_PER_GROUP


@functools.lru_cache(maxsize=None)
def _make_kernel(B):
    rows_per_w = B // NUM_WORKERS
    groups_per_w = rows_per_w // GROUP
    mesh = plsc.VectorSubcoreMesh(core_axis_name="c", subcore_axis_name="s")

    @functools.partial(
        pl.kernel,
        out_type=jax.ShapeDtypeStruct((B, D_MODEL), jnp.float32),
        mesh=mesh,
        scratch_types=[
            pltpu.VMEM((GATHERS_PER_GROUP, ROWS_PER_GATHER), jnp.int32),
            pltpu.VMEM((GROUP, D_MODEL), jnp.float32),
            pltpu.SemaphoreType.DMA,
        ],
    )
    def gather_kernel(idx_hbm, table_hbm, out_hbm, idx_v, rows_v, sem):
        wid = lax.axis_index("s") * 2 + lax.axis_index("c")
        w_base = wid * rows_per_w

        def body(g, carry):
            gbase = w_base + g * GROUP
            pltpu.sync_copy(
                idx_hbm.at[pl.ds(gbase // ROWS_PER_GATHER, GATHERS_PER_GROUP), :],
                idx_v)
            copies = []
            for j in range(GATHERS_PER_GROUP):
                copies.append(pltpu.async_copy(
                    table_hbm.at[idx_v.at[j]],
                    rows_v.at[pl.ds(j * ROWS_PER_GATHER, ROWS_PER_GATHER)],
                    sem))
            for c in copies:
                c.wait()
            pltpu.sync_copy(rows_v, out_hbm.at[pl.ds(gbase, GROUP)])
            return carry

        lax.fori_loop(0, groups_per_w, body, 0)

    return gather_kernel


def kernel(tf_indices, table):
    batch, hist = tf_indices.shape
    B = batch * hist
    idx2 = tf_indices.reshape(B // ROWS_PER_GATHER, ROWS_PER_GATHER)
    idx2 = idx2.astype(jnp.int32)
    out = _make_kernel(B)(idx2, table)
    return out.reshape(batch, hist, D_MODEL)


# speedup vs baseline: 4.1432x; 4.1432x over previous
import functools

import jax
import jax.numpy as jnp
from jax import lax
from jax.experimental import pallas as pl
from jax.experimental.pallas import tpu as pltpu
from jax.experimental.pallas import tpu_sc as plsc

D_MODEL = 64
NUM_WORKERS = 32
ROWS_PER_GATHER = 128
GATHERS_PER_GROUP = 8
GROUP = ROWS_PER_GATHER * GATHERS_PER_GROUP


@functools.lru_cache(maxsize=None)
def _make_kernel(B):
    rows_per_w = B // NUM_WORKERS
    groups_per_w = rows_per_w // GROUP
    mesh = plsc.VectorSubcoreMesh(core_axis_name="c", subcore_axis_name="s")

    @functools.partial(
        pl.kernel,
        out_type=jax.ShapeDtypeStruct((B, D_MODEL), jnp.float32),
        mesh=mesh,
        scratch_types=[
            pltpu.VMEM((GATHERS_PER_GROUP, ROWS_PER_GATHER), jnp.int32),
            pltpu.VMEM((GROUP, D_MODEL), jnp.float32),
            pltpu.SemaphoreType.DMA,
        ],
        compiler_params=pltpu.CompilerParams(use_tc_tiling_on_sc=False),
    )
    def gather_kernel(idx_hbm, table_hbm, out_hbm, idx_v, rows_v, sem):
        wid = lax.axis_index("s") * 2 + lax.axis_index("c")
        w_base = wid * rows_per_w

        def body(g, carry):
            gbase = pl.multiple_of(w_base + g * GROUP, GROUP)
            irow = pl.multiple_of(gbase // ROWS_PER_GATHER, GATHERS_PER_GROUP)
            pltpu.sync_copy(
                idx_hbm.at[pl.ds(irow, GATHERS_PER_GROUP), :],
                idx_v)
            copies = []
            for j in range(GATHERS_PER_GROUP):
                copies.append(pltpu.async_copy(
                    table_hbm.at[idx_v.at[j]],
                    rows_v.at[pl.ds(j * ROWS_PER_GATHER, ROWS_PER_GATHER)],
                    sem))
            for c in copies:
                c.wait()
            pltpu.sync_copy(rows_v, out_hbm.at[pl.ds(gbase, GROUP)])
            return carry

        lax.fori_loop(0, groups_per_w, body, 0)

    return gather_kernel


def kernel(tf_indices, table):
    batch, hist = tf_indices.shape
    B = batch * hist
    idx2 = tf_indices.reshape(B // ROWS_PER_GATHER, ROWS_PER_GATHER)
    idx2 = idx2.astype(jnp.int32)
    out = _make_kernel(B)(idx2, table)
    return out.reshape(batch, hist, D_MODEL)
